# Initial kernel scaffold; baseline (speedup 1.0000x reference)
import functools

import jax
import jax.numpy as jnp
from jax import lax
from jax.experimental import pallas as pl
from jax.experimental.pallas import tpu as pltpu
from jax.experimental.pallas import tpu_sc as plsc

N = 10000
E = 320000
IN_FEATS = 128
N_CLASSES = 16
ALPHA = 0.1
NITER = 10

NP_ = 10240
CH = 128
TILES = 16
EPT = 20096
CPT = EPT // CH
NEP = TILES * EPT
RPT = NP_ // 32
RPS = NP_ // TILES
MLP_BM = 256

_mesh = plsc.VectorSubcoreMesh(core_axis_name="c", subcore_axis_name="s")

_SC_SCRATCH = [
    pltpu.VMEM_SHARED((NP_, N_CLASSES), jnp.float32),
    pltpu.VMEM((CH,), jnp.int32),
    pltpu.VMEM((CH,), jnp.int32),
    pltpu.VMEM((CH, N_CLASSES), jnp.float32),
    pltpu.VMEM((RPT, N_CLASSES), jnp.float32),
    pltpu.VMEM((RPT, N_CLASSES), jnp.float32),
    pltpu.VMEM((RPT, N_CLASSES), jnp.float32),
    pltpu.VMEM((RPT, N_CLASSES), jnp.float32),
    pltpu.SemaphoreType.DMA,
]


def _zero_vmem(ref, nrows):
    def body(i, _):
        ref[i] = jnp.zeros((N_CLASSES,), jnp.float32)
        return 0
    lax.fori_loop(0, nrows, body, 0)


@functools.partial(
    pl.kernel,
    out_type=jax.ShapeDtypeStruct((NP_, N_CLASSES), jnp.float32),
    mesh=_mesh,
    scratch_types=_SC_SCRATCH,
)
def _deg_step(row_hbm, ones_hbm, deg_hbm, agg_sh, colv, rowv, buf,
              acc_v, u_v, d2_v, la_v, sem):
    del colv, u_v, d2_v, la_v, sem
    c = lax.axis_index("c")
    s = lax.axis_index("s")

    _zero_vmem(acc_v, RPT)
    pltpu.sync_copy(acc_v, agg_sh.at[pl.ds(s * RPS, RPT)])
    pltpu.sync_copy(acc_v, agg_sh.at[pl.ds(s * RPS + RPT, RPT)])
    pltpu.sync_copy(ones_hbm, buf)
    plsc.subcore_barrier()

    def edge_body(k, _):
        base = s * EPT + k * CH
        pltpu.sync_copy(row_hbm.at[pl.ds(base, CH)], rowv)
        pltpu.sync_copy(buf, agg_sh.at[rowv], add=True)
        return 0
    lax.fori_loop(0, CPT, edge_body, 0)

    plsc.subcore_barrier()

    base = (c * TILES + s) * RPT
    pltpu.sync_copy(agg_sh.at[pl.ds(base, RPT)], acc_v)
    pltpu.sync_copy(acc_v, deg_hbm.at[pl.ds(base, RPT)])


@functools.partial(
    pl.kernel,
    out_type=jax.ShapeDtypeStruct((NP_, N_CLASSES), jnp.float32),
    mesh=_mesh,
    scratch_types=_SC_SCRATCH,
)
def _iter_step(u_hbm, col_hbm, row_hbm, d2_hbm, la_hbm, out_hbm,
               agg_sh, colv, rowv, buf, acc_v, u_v, d2_v, la_v, sem):
    c = lax.axis_index("c")
    s = lax.axis_index("s")

    _zero_vmem(acc_v, RPT)
    pltpu.sync_copy(acc_v, agg_sh.at[pl.ds(s * RPS, RPT)])
    pltpu.sync_copy(acc_v, agg_sh.at[pl.ds(s * RPS + RPT, RPT)])
    plsc.subcore_barrier()

    def edge_body(k, _):
        base = s * EPT + k * CH
        pltpu.sync_copy(col_hbm.at[pl.ds(base, CH)], colv)
        pltpu.async_copy(u_hbm.at[colv], buf, sem).wait()
        pltpu.sync_copy(row_hbm.at[pl.ds(base, CH)], rowv)
        pltpu.sync_copy(buf, agg_sh.at[rowv], add=True)
        return 0
    lax.fori_loop(0, CPT, edge_body, 0)

    plsc.subcore_barrier()

    base = (c * TILES + s) * RPT
    pltpu.sync_copy(agg_sh.at[pl.ds(base, RPT)], acc_v)
    pltpu.sync_copy(u_hbm.at[pl.ds(base, RPT)], u_v)
    pltpu.sync_copy(d2_hbm.at[pl.ds(base, RPT)], d2_v)
    pltpu.sync_copy(la_hbm.at[pl.ds(base, RPT)], la_v)

    def fin_body(r, _):
        acc_v[r] = d2_v[r] * (acc_v[r] + u_v[r]) + la_v[r]
        return 0
    lax.fori_loop(0, RPT, fin_body, 0)

    pltpu.sync_copy(acc_v, out_hbm.at[pl.ds(base, RPT)])


def _mlp_body(x_ref, w1_ref, w2_ref, deg_ref, u0_ref, d2_ref, la_ref,
              d2l_ref, lal_ref):
    pid = pl.program_id(0)
    h = jnp.tanh(jnp.dot(x_ref[...], w1_ref[...],
                         preferred_element_type=jnp.float32))
    local = jnp.dot(h, w2_ref[...], preferred_element_type=jnp.float32)
    deg = deg_ref[:, 0:1] + 1.0
    dinv = lax.rsqrt(jnp.maximum(deg, 1e-12))
    rows = pid * MLP_BM + lax.broadcasted_iota(jnp.int32, (MLP_BM, 1), 0)
    dinv = jnp.where(rows < N, dinv, 0.0)
    scale = 1.0 - ALPHA
    u0_ref[...] = dinv * local
    d2_ref[...] = jnp.broadcast_to(scale * dinv * dinv, (MLP_BM, N_CLASSES))
    la_ref[...] = ALPHA * dinv * local
    d2l_ref[...] = jnp.broadcast_to(scale * dinv, (MLP_BM, N_CLASSES))
    lal_ref[...] = ALPHA * local


_out16 = jax.ShapeDtypeStruct((NP_, N_CLASSES), jnp.float32)

_mlp = pl.pallas_call(
    _mlp_body,
    grid=(NP_ // MLP_BM,),
    in_specs=[
        pl.BlockSpec((MLP_BM, IN_FEATS), lambda i: (i, 0)),
        pl.BlockSpec((IN_FEATS, 64), lambda i: (0, 0)),
        pl.BlockSpec((64, N_CLASSES), lambda i: (0, 0)),
        pl.BlockSpec((MLP_BM, N_CLASSES), lambda i: (i, 0)),
    ],
    out_specs=[pl.BlockSpec((MLP_BM, N_CLASSES), lambda i: (i, 0))] * 5,
    out_shape=[_out16] * 5,
)


def kernel(local_preds, edge_index, W1, W2):
    x = jnp.pad(local_preds, ((0, NP_ - N), (0, 0)))
    pad_idx = jnp.full((NEP - E,), NP_ - 1, jnp.int32)
    row = jnp.concatenate([edge_index[0], pad_idx])
    col = jnp.concatenate([edge_index[1], pad_idx])
    ones = jnp.ones((CH, N_CLASSES), jnp.float32)

    deg = _deg_step(row, ones)
    u0, d2, la, d2l, lal = _mlp(x, W1, W2, deg)

    u = lax.fori_loop(
        0, NITER - 1, lambda i, u: _iter_step(u, col, row, d2, la), u0)
    preds = _iter_step(u, col, row, d2l, lal)
    return preds[:N]


# speedup vs baseline: 7.9373x; 7.9373x over previous
import functools

import jax
import jax.numpy as jnp
from jax import lax
from jax.experimental import pallas as pl
from jax.experimental.pallas import tpu as pltpu
from jax.experimental.pallas import tpu_sc as plsc

N = 10000
E = 320000
IN_FEATS = 128
N_CLASSES = 16
ALPHA = 0.1
NITER = 10

NP_ = 10240
CH = 128
TILES = 16
EPT = 20096
CPT = EPT // CH
NEP = TILES * EPT
RPT = NP_ // 32
RPS = NP_ // TILES
MLP_BM = 256

_mesh = plsc.VectorSubcoreMesh(core_axis_name="c", subcore_axis_name="s")

_SC_SCRATCH = [
    pltpu.VMEM_SHARED((NP_, N_CLASSES), jnp.float32),
    pltpu.VMEM((CH,), jnp.int32),
    pltpu.VMEM((CH,), jnp.int32),
    pltpu.VMEM((CH, N_CLASSES), jnp.float32),
    pltpu.VMEM((RPT, N_CLASSES), jnp.float32),
    pltpu.VMEM((RPT, N_CLASSES), jnp.float32),
    pltpu.VMEM((RPT, N_CLASSES), jnp.float32),
    pltpu.VMEM((RPT, N_CLASSES), jnp.float32),
    pltpu.SemaphoreType.DMA,
]


def _zero_vmem(ref, nrows):
    def body(i, _):
        ref[i] = jnp.zeros((N_CLASSES,), jnp.float32)
        return 0
    lax.fori_loop(0, nrows, body, 0)


@functools.partial(
    pl.kernel,
    out_type=jax.ShapeDtypeStruct((NP_, N_CLASSES), jnp.float32),
    mesh=_mesh,
    scratch_types=_SC_SCRATCH,
    compiler_params=pltpu.CompilerParams(use_tc_tiling_on_sc=False),
)
def _deg_step(row_hbm, ones_hbm, deg_hbm, agg_sh, colv, rowv, buf,
              acc_v, u_v, d2_v, la_v, sem):
    del colv, u_v, d2_v, la_v, sem
    c = lax.axis_index("c")
    s = lax.axis_index("s")

    _zero_vmem(acc_v, RPT)
    pltpu.sync_copy(acc_v, agg_sh.at[pl.ds(s * RPS, RPT)])
    pltpu.sync_copy(acc_v, agg_sh.at[pl.ds(s * RPS + RPT, RPT)])
    pltpu.sync_copy(ones_hbm, buf)
    plsc.subcore_barrier()

    def edge_body(k, _):
        base = s * EPT + k * CH
        pltpu.sync_copy(row_hbm.at[pl.ds(base, CH)], rowv)
        pltpu.sync_copy(buf, agg_sh.at[rowv], add=True)
        return 0
    lax.fori_loop(0, CPT, edge_body, 0)

    plsc.subcore_barrier()

    base = (c * TILES + s) * RPT
    pltpu.sync_copy(agg_sh.at[pl.ds(base, RPT)], acc_v)
    pltpu.sync_copy(acc_v, deg_hbm.at[pl.ds(base, RPT)])


@functools.partial(
    pl.kernel,
    out_type=jax.ShapeDtypeStruct((NP_, N_CLASSES), jnp.float32),
    mesh=_mesh,
    scratch_types=_SC_SCRATCH,
    compiler_params=pltpu.CompilerParams(use_tc_tiling_on_sc=False),
)
def _iter_step(u_hbm, col_hbm, row_hbm, d2_hbm, la_hbm, out_hbm,
               agg_sh, colv, rowv, buf, acc_v, u_v, d2_v, la_v, sem):
    c = lax.axis_index("c")
    s = lax.axis_index("s")

    _zero_vmem(acc_v, RPT)
    pltpu.sync_copy(acc_v, agg_sh.at[pl.ds(s * RPS, RPT)])
    pltpu.sync_copy(acc_v, agg_sh.at[pl.ds(s * RPS + RPT, RPT)])
    plsc.subcore_barrier()

    def edge_body(k, _):
        base = s * EPT + k * CH
        pltpu.sync_copy(col_hbm.at[pl.ds(base, CH)], colv)
        pltpu.async_copy(u_hbm.at[colv], buf, sem).wait()
        pltpu.sync_copy(row_hbm.at[pl.ds(base, CH)], rowv)
        pltpu.sync_copy(buf, agg_sh.at[rowv], add=True)
        return 0
    lax.fori_loop(0, CPT, edge_body, 0)

    plsc.subcore_barrier()

    base = (c * TILES + s) * RPT
    pltpu.sync_copy(agg_sh.at[pl.ds(base, RPT)], acc_v)
    pltpu.sync_copy(u_hbm.at[pl.ds(base, RPT)], u_v)
    pltpu.sync_copy(d2_hbm.at[pl.ds(base, RPT)], d2_v)
    pltpu.sync_copy(la_hbm.at[pl.ds(base, RPT)], la_v)

    def fin_body(r, _):
        acc_v[r] = d2_v[r] * (acc_v[r] + u_v[r]) + la_v[r]
        return 0
    lax.fori_loop(0, RPT, fin_body, 0)

    pltpu.sync_copy(acc_v, out_hbm.at[pl.ds(base, RPT)])


def _mlp_body(x_ref, w1_ref, w2_ref, deg_ref, u0_ref, d2_ref, la_ref,
              d2l_ref, lal_ref):
    pid = pl.program_id(0)
    h = jnp.tanh(jnp.dot(x_ref[...], w1_ref[...],
                         preferred_element_type=jnp.float32))
    local = jnp.dot(h, w2_ref[...], preferred_element_type=jnp.float32)
    deg = deg_ref[:, 0:1] + 1.0
    dinv = lax.rsqrt(jnp.maximum(deg, 1e-12))
    rows = pid * MLP_BM + lax.broadcasted_iota(jnp.int32, (MLP_BM, 1), 0)
    dinv = jnp.where(rows < N, dinv, 0.0)
    scale = 1.0 - ALPHA
    u0_ref[...] = dinv * local
    d2_ref[...] = jnp.broadcast_to(scale * dinv * dinv, (MLP_BM, N_CLASSES))
    la_ref[...] = ALPHA * dinv * local
    d2l_ref[...] = jnp.broadcast_to(scale * dinv, (MLP_BM, N_CLASSES))
    lal_ref[...] = ALPHA * local


_out16 = jax.ShapeDtypeStruct((NP_, N_CLASSES), jnp.float32)

_mlp = pl.pallas_call(
    _mlp_body,
    grid=(NP_ // MLP_BM,),
    in_specs=[
        pl.BlockSpec((MLP_BM, IN_FEATS), lambda i: (i, 0)),
        pl.BlockSpec((IN_FEATS, 64), lambda i: (0, 0)),
        pl.BlockSpec((64, N_CLASSES), lambda i: (0, 0)),
        pl.BlockSpec((MLP_BM, N_CLASSES), lambda i: (i, 0)),
    ],
    out_specs=[pl.BlockSpec((MLP_BM, N_CLASSES), lambda i: (i, 0))] * 5,
    out_shape=[_out16] * 5,
)


def kernel(local_preds, edge_index, W1, W2):
    x = jnp.pad(local_preds, ((0, NP_ - N), (0, 0)))
    pad_idx = jnp.full((NEP - E,), NP_ - 1, jnp.int32)
    row = jnp.concatenate([edge_index[0], pad_idx])
    col = jnp.concatenate([edge_index[1], pad_idx])
    ones = jnp.ones((CH, N_CLASSES), jnp.float32)

    deg = _deg_step(row, ones)
    u0, d2, la, d2l, lal = _mlp(x, W1, W2, deg)

    u = lax.fori_loop(
        0, NITER - 1, lambda i, u: _iter_step(u, col, row, d2, la), u0)
    preds = _iter_step(u, col, row, d2l, lal)
    return preds[:N]


# speedup vs baseline: 31.5016x; 3.9688x over previous
import functools

import jax
import jax.numpy as jnp
from jax import lax
from jax.experimental import pallas as pl
from jax.experimental.pallas import tpu as pltpu
from jax.experimental.pallas import tpu_sc as plsc

N = 10000
E = 320000
IN_FEATS = 128
N_CLASSES = 16
ALPHA = 0.1
NITER = 10

NP_ = 10240
CH = 128
TILES = 16
EPT = 20096
CPT = EPT // CH
NEP = TILES * EPT
RPT = NP_ // 32
RPS = NP_ // TILES
MLP_BM = 256

_mesh = plsc.VectorSubcoreMesh(core_axis_name="c", subcore_axis_name="s")

_SC_SCRATCH = [
    pltpu.VMEM_SHARED((NP_, N_CLASSES), jnp.float32),
    pltpu.VMEM((CH,), jnp.int32),
    pltpu.VMEM((CH,), jnp.int32),
    pltpu.VMEM((CH, N_CLASSES), jnp.float32),
    pltpu.VMEM((RPT, N_CLASSES), jnp.float32),
    pltpu.VMEM((RPT, N_CLASSES), jnp.float32),
    pltpu.VMEM((RPT, N_CLASSES), jnp.float32),
    pltpu.VMEM((RPT, N_CLASSES), jnp.float32),
    pltpu.SemaphoreType.DMA,
]


def _zero_vmem(ref, nrows):
    def body(i, _):
        ref[i] = jnp.zeros((N_CLASSES,), jnp.float32)
        return 0
    lax.fori_loop(0, nrows, body, 0)


@functools.partial(
    pl.kernel,
    out_type=jax.ShapeDtypeStruct((NP_, N_CLASSES), jnp.float32),
    mesh=_mesh,
    scratch_types=_SC_SCRATCH,
    compiler_params=pltpu.CompilerParams(use_tc_tiling_on_sc=False),
)
def _deg_step(row_hbm, ones_hbm, deg_hbm, agg_sh, colv, rowv, buf,
              acc_v, u_v, d2_v, la_v, sem):
    del colv, u_v, d2_v, la_v, sem
    c = lax.axis_index("c")
    s = lax.axis_index("s")

    _zero_vmem(acc_v, RPT)
    pltpu.sync_copy(acc_v, agg_sh.at[pl.ds(s * RPS, RPT)])
    pltpu.sync_copy(acc_v, agg_sh.at[pl.ds(s * RPS + RPT, RPT)])
    pltpu.sync_copy(ones_hbm, buf)
    plsc.subcore_barrier()

    def edge_body(k, _):
        base = s * EPT + k * CH
        pltpu.sync_copy(row_hbm.at[pl.ds(base, CH)], rowv)
        pltpu.sync_copy(buf, agg_sh.at[rowv], add=True)
        return 0
    lax.fori_loop(0, CPT, edge_body, 0)

    plsc.subcore_barrier()

    base = (c * TILES + s) * RPT
    pltpu.sync_copy(agg_sh.at[pl.ds(base, RPT)], acc_v)
    pltpu.sync_copy(acc_v, deg_hbm.at[pl.ds(base, RPT)])


@functools.partial(
    pl.kernel,
    out_type=jax.ShapeDtypeStruct((NP_, N_CLASSES), jnp.float32),
    mesh=_mesh,
    scratch_types=[
        pltpu.VMEM_SHARED((NP_, N_CLASSES), jnp.float32),
        pltpu.VMEM_SHARED((NP_, N_CLASSES), jnp.float32),
        pltpu.VMEM((CPT, CH), jnp.int32),
        pltpu.VMEM((CPT, CH), jnp.int32),
        pltpu.VMEM((CH, N_CLASSES), jnp.float32),
        pltpu.VMEM((CH, N_CLASSES), jnp.float32),
        pltpu.VMEM((RPS, N_CLASSES), jnp.float32),
        pltpu.VMEM((RPS, N_CLASSES), jnp.float32),
        pltpu.VMEM((RPS, N_CLASSES), jnp.float32),
        pltpu.VMEM((RPS, N_CLASSES), jnp.float32),
        pltpu.VMEM((RPS, N_CLASSES), jnp.float32),
        pltpu.VMEM((RPS, N_CLASSES), jnp.float32),
        pltpu.SemaphoreType.DMA,
    ],
    compiler_params=pltpu.CompilerParams(use_tc_tiling_on_sc=False),
)
def _power(u0_hbm, col_hbm, row_hbm, d2_hbm, la_hbm, d2l_hbm, lal_hbm,
           out_hbm, u_sh, agg_sh, colv, rowv, buf, zbuf, agg_v, u_v,
           d2_v, la_v, d2l_v, lal_v, sem):
    c = lax.axis_index("c")
    s = lax.axis_index("s")
    mine = s * RPS

    pltpu.sync_copy(col_hbm.at[s], colv)
    pltpu.sync_copy(row_hbm.at[s], rowv)
    pltpu.sync_copy(d2_hbm.at[pl.ds(mine, RPS)], d2_v)
    pltpu.sync_copy(la_hbm.at[pl.ds(mine, RPS)], la_v)
    pltpu.sync_copy(d2l_hbm.at[pl.ds(mine, RPS)], d2l_v)
    pltpu.sync_copy(lal_hbm.at[pl.ds(mine, RPS)], lal_v)
    _zero_vmem(zbuf, CH)

    def _zero_agg():
        def zb(j, _):
            pltpu.sync_copy(zbuf, agg_sh.at[pl.ds(mine + j * CH, CH)])
            return 0
        lax.fori_loop(0, RPS // CH, zb, 0)

    pltpu.sync_copy(u0_hbm.at[pl.ds(mine, RPS)], u_v)
    pltpu.sync_copy(u_v, u_sh.at[pl.ds(mine, RPS)])
    _zero_agg()
    plsc.subcore_barrier()

    def one_iter(t, _):
        def edge_body(k, _):
            pltpu.async_copy(u_sh.at[colv.at[k]], buf, sem).wait()
            pltpu.sync_copy(buf, agg_sh.at[rowv.at[k]], add=True)
            return 0
        lax.fori_loop(0, CPT, edge_body, 0)
        plsc.subcore_barrier()

        pltpu.sync_copy(agg_sh.at[pl.ds(mine, RPS)], agg_v)
        pltpu.sync_copy(u_sh.at[pl.ds(mine, RPS)], u_v)
        _zero_agg()
        last = t == NITER - 1

        def fin_body(r, _):
            d2r = jnp.where(last, d2l_v[r], d2_v[r])
            lar = jnp.where(last, lal_v[r], la_v[r])
            u_v[r] = d2r * (agg_v[r] + u_v[r]) + lar
            return 0
        lax.fori_loop(0, RPS, fin_body, 0)
        pltpu.sync_copy(u_v, u_sh.at[pl.ds(mine, RPS)])
        plsc.subcore_barrier()
        return 0

    lax.fori_loop(0, NITER, one_iter, 0)

    base = c * (NP_ // 2) + s * RPT
    pltpu.sync_copy(u_sh.at[pl.ds(base, RPT)], agg_v.at[pl.ds(0, RPT)])
    pltpu.sync_copy(agg_v.at[pl.ds(0, RPT)], out_hbm.at[pl.ds(base, RPT)])


def _mlp_body(x_ref, w1_ref, w2_ref, deg_ref, u0_ref, d2_ref, la_ref,
              d2l_ref, lal_ref):
    pid = pl.program_id(0)
    h = jnp.tanh(jnp.dot(x_ref[...], w1_ref[...],
                         preferred_element_type=jnp.float32))
    local = jnp.dot(h, w2_ref[...], preferred_element_type=jnp.float32)
    deg = deg_ref[:, 0:1] + 1.0
    dinv = lax.rsqrt(jnp.maximum(deg, 1e-12))
    rows = pid * MLP_BM + lax.broadcasted_iota(jnp.int32, (MLP_BM, 1), 0)
    dinv = jnp.where(rows < N, dinv, 0.0)
    scale = 1.0 - ALPHA
    u0_ref[...] = dinv * local
    d2_ref[...] = jnp.broadcast_to(scale * dinv * dinv, (MLP_BM, N_CLASSES))
    la_ref[...] = ALPHA * dinv * local
    d2l_ref[...] = jnp.broadcast_to(scale * dinv, (MLP_BM, N_CLASSES))
    lal_ref[...] = ALPHA * local


_out16 = jax.ShapeDtypeStruct((NP_, N_CLASSES), jnp.float32)

_mlp = pl.pallas_call(
    _mlp_body,
    grid=(NP_ // MLP_BM,),
    in_specs=[
        pl.BlockSpec((MLP_BM, IN_FEATS), lambda i: (i, 0)),
        pl.BlockSpec((IN_FEATS, 64), lambda i: (0, 0)),
        pl.BlockSpec((64, N_CLASSES), lambda i: (0, 0)),
        pl.BlockSpec((MLP_BM, N_CLASSES), lambda i: (i, 0)),
    ],
    out_specs=[pl.BlockSpec((MLP_BM, N_CLASSES), lambda i: (i, 0))] * 5,
    out_shape=[_out16] * 5,
)


def kernel(local_preds, edge_index, W1, W2):
    x = jnp.pad(local_preds, ((0, NP_ - N), (0, 0)))
    pad_idx = jnp.full((NEP - E,), NP_ - 1, jnp.int32)
    row = jnp.concatenate([edge_index[0], pad_idx])
    col = jnp.concatenate([edge_index[1], pad_idx])
    ones = jnp.ones((CH, N_CLASSES), jnp.float32)

    deg = _deg_step(row, ones)
    u0, d2, la, d2l, lal = _mlp(x, W1, W2, deg)

    col3 = col.reshape(TILES, CPT, CH)
    row3 = row.reshape(TILES, CPT, CH)
    preds = _power(u0, col3, row3, d2, la, d2l, lal)
    return preds[:N]


# speedup vs baseline: 45.0522x; 1.4302x over previous
import functools

import jax
import jax.numpy as jnp
from jax import lax
from jax.experimental import pallas as pl
from jax.experimental.pallas import tpu as pltpu
from jax.experimental.pallas import tpu_sc as plsc

N = 10000
E = 320000
IN_FEATS = 128
N_CLASSES = 16
ALPHA = 0.1
NITER = 10

NP_ = 10240
CH = 128
TILES = 16
CPT = 160
EPT = CPT * CH
NEP = TILES * EPT
RPT = NP_ // 32
RPS = NP_ // TILES
NB = 4
NBATCH = CPT // NB
MLP_BM = 256

_mesh = plsc.VectorSubcoreMesh(core_axis_name="c", subcore_axis_name="s")
_params = pltpu.CompilerParams(use_tc_tiling_on_sc=False)
_out16 = jax.ShapeDtypeStruct((NP_, N_CLASSES), jnp.float32)


def _zero_vmem(ref, nrows):
    def body(i, _):
        ref[i] = jnp.zeros((N_CLASSES,), jnp.float32)
        return 0
    lax.fori_loop(0, nrows, body, 0)


@functools.partial(
    pl.kernel,
    out_type=_out16,
    mesh=_mesh,
    scratch_types=[
        pltpu.VMEM_SHARED((NP_, N_CLASSES), jnp.float32),
        pltpu.VMEM((CPT, CH), jnp.int32),
        pltpu.VMEM((CH, N_CLASSES), jnp.float32),
        pltpu.VMEM((CH, N_CLASSES), jnp.float32),
        pltpu.VMEM((RPT, N_CLASSES), jnp.float32),
        pltpu.SemaphoreType.DMA,
    ],
    compiler_params=_params,
)
def _deg_step(row_hbm, deg_hbm, deg_sh, rowv, onev, zbuf, out_v, sem):
    c = lax.axis_index("c")
    s = lax.axis_index("s")
    mine = s * RPS

    pltpu.sync_copy(row_hbm.at[s], rowv)
    _zero_vmem(zbuf, CH)

    def ob(i, _):
        onev[i] = jnp.full((N_CLASSES,), 1.0, jnp.float32)
        return 0
    lax.fori_loop(0, CH, ob, 0)

    def zb(j, _):
        pltpu.sync_copy(zbuf, deg_sh.at[pl.ds(mine + j * CH, CH)])
        return 0
    lax.fori_loop(0, RPS // CH, zb, 0)
    plsc.subcore_barrier()

    def batch(i, _):
        for b in range(NB):
            pltpu.async_copy(onev, deg_sh.at[rowv.at[i * NB + b]], sem,
                             add=True)

        @pl.when(i >= 1)
        def _():
            for _b in range(NB):
                pltpu.make_async_copy(onev, deg_sh.at[rowv.at[0]], sem).wait()
        return 0
    lax.fori_loop(0, NBATCH, batch, 0)
    for _b in range(NB):
        pltpu.make_async_copy(onev, deg_sh.at[rowv.at[0]], sem).wait()

    plsc.subcore_barrier()
    base = (c * TILES + s) * RPT
    pltpu.sync_copy(deg_sh.at[pl.ds(base, RPT)], out_v)
    pltpu.sync_copy(out_v, deg_hbm.at[pl.ds(base, RPT)])


@functools.partial(
    pl.kernel,
    out_type=_out16,
    mesh=_mesh,
    scratch_types=[
        pltpu.VMEM_SHARED((NP_, N_CLASSES), jnp.float32),
        pltpu.VMEM_SHARED((NP_, N_CLASSES), jnp.float32),
        pltpu.VMEM((CPT, CH), jnp.int32),
        pltpu.VMEM((CPT, CH), jnp.int32),
        pltpu.VMEM((2 * NB, CH, N_CLASSES), jnp.float32),
        pltpu.VMEM((CH, N_CLASSES), jnp.float32),
        pltpu.VMEM((RPS, N_CLASSES), jnp.float32),
        pltpu.VMEM((RPS, N_CLASSES), jnp.float32),
        pltpu.VMEM((RPS, N_CLASSES), jnp.float32),
        pltpu.VMEM((RPS, N_CLASSES), jnp.float32),
        pltpu.SemaphoreType.DMA,
        pltpu.SemaphoreType.DMA,
        pltpu.SemaphoreType.DMA,
    ],
    compiler_params=_params,
)
def _power(u0_hbm, col_hbm, row_hbm, d2_hbm, la_hbm, d2l_hbm, lal_hbm,
           out_hbm, u_sh, agg_sh, colv, rowv, ring, zbuf, agg_v, u_v,
           d2_v, la_v, semg0, semg1, sems):
    c = lax.axis_index("c")
    s = lax.axis_index("s")
    mine = s * RPS

    pltpu.sync_copy(col_hbm.at[s], colv)
    pltpu.sync_copy(row_hbm.at[s], rowv)
    pltpu.sync_copy(d2_hbm.at[pl.ds(mine, RPS)], d2_v)
    pltpu.sync_copy(la_hbm.at[pl.ds(mine, RPS)], la_v)
    _zero_vmem(zbuf, CH)

    def _zero_agg():
        def zb(j, _):
            pltpu.sync_copy(zbuf, agg_sh.at[pl.ds(mine + j * CH, CH)])
            return 0
        lax.fori_loop(0, RPS // CH, zb, 0)

    pltpu.sync_copy(u0_hbm.at[pl.ds(mine, RPS)], u_v)
    pltpu.sync_copy(u_v, u_sh.at[pl.ds(mine, RPS)])
    _zero_agg()
    plsc.subcore_barrier()

    def _fire_gathers(base_k, slot, sem):
        for b in range(NB):
            pltpu.async_copy(u_sh.at[colv.at[base_k + b]],
                             ring.at[slot + b], sem)

    def _drain(n, sem):
        for _b in range(n):
            pltpu.make_async_copy(u_sh.at[colv.at[0]], ring.at[0], sem).wait()

    def one_iter(t, _):
        _fire_gathers(0, 0, semg0)

        def batch2(i, _):
            par = i % 2
            h = par * NB
            h2 = NB - h

            @pl.when(i >= 1)
            def _():
                _drain(NB, sems)

            @pl.when((i + 1 < NBATCH) & (par == 0))
            def _():
                _fire_gathers((i + 1) * NB, NB, semg1)

            @pl.when((i + 1 < NBATCH) & (par == 1))
            def _():
                _fire_gathers((i + 1) * NB, 0, semg0)

            @pl.when(par == 0)
            def _():
                _drain(NB, semg0)

            @pl.when(par == 1)
            def _():
                _drain(NB, semg1)

            for b in range(NB):
                pltpu.async_copy(ring.at[h + b],
                                 agg_sh.at[rowv.at[i * NB + b]], sems,
                                 add=True)
            return 0

        lax.fori_loop(0, NBATCH, batch2, 0)
        _drain(NB, sems)
        plsc.subcore_barrier()

        pltpu.sync_copy(agg_sh.at[pl.ds(mine, RPS)], agg_v)

        @pl.when(t == NITER - 1)
        def _():
            pltpu.sync_copy(d2l_hbm.at[pl.ds(mine, RPS)], d2_v)
            pltpu.sync_copy(lal_hbm.at[pl.ds(mine, RPS)], la_v)

        def fin_body(r, _):
            r4 = r * 4
            for j in range(4):
                u_v[r4 + j] = (d2_v[r4 + j] * (agg_v[r4 + j] + u_v[r4 + j])
                               + la_v[r4 + j])
            return 0
        lax.fori_loop(0, RPS // 4, fin_body, 0)
        _zero_agg()
        pltpu.sync_copy(u_v, u_sh.at[pl.ds(mine, RPS)])
        plsc.subcore_barrier()
        return 0

    lax.fori_loop(0, NITER, one_iter, 0)

    base = c * (NP_ // 2) + s * RPT
    pltpu.sync_copy(u_sh.at[pl.ds(base, RPT)], agg_v.at[pl.ds(0, RPT)])
    pltpu.sync_copy(agg_v.at[pl.ds(0, RPT)], out_hbm.at[pl.ds(base, RPT)])


def _local_body(x_ref, w1_ref, w2_ref, local_ref):
    h = jnp.tanh(jnp.dot(x_ref[...], w1_ref[...],
                         preferred_element_type=jnp.float32))
    local_ref[...] = jnp.dot(h, w2_ref[...],
                             preferred_element_type=jnp.float32)


_local = pl.pallas_call(
    _local_body,
    grid=(NP_ // MLP_BM,),
    in_specs=[
        pl.BlockSpec((MLP_BM, IN_FEATS), lambda i: (i, 0)),
        pl.BlockSpec((IN_FEATS, 64), lambda i: (0, 0)),
        pl.BlockSpec((64, N_CLASSES), lambda i: (0, 0)),
    ],
    out_specs=pl.BlockSpec((MLP_BM, N_CLASSES), lambda i: (i, 0)),
    out_shape=_out16,
)


def _scales_body(local_ref, deg_ref, u0_ref, d2_ref, la_ref, d2l_ref,
                 lal_ref):
    pid = pl.program_id(0)
    local = local_ref[...]
    deg = deg_ref[:, 0:1] + 1.0
    dinv = lax.rsqrt(jnp.maximum(deg, 1e-12))
    rows = pid * MLP_BM + lax.broadcasted_iota(jnp.int32, (MLP_BM, 1), 0)
    dinv = jnp.where(rows < N, dinv, 0.0)
    scale = 1.0 - ALPHA
    u0_ref[...] = dinv * local
    d2_ref[...] = jnp.broadcast_to(scale * dinv * dinv, (MLP_BM, N_CLASSES))
    la_ref[...] = ALPHA * dinv * local
    d2l_ref[...] = jnp.broadcast_to(scale * dinv, (MLP_BM, N_CLASSES))
    lal_ref[...] = ALPHA * local


_scales = pl.pallas_call(
    _scales_body,
    grid=(NP_ // MLP_BM,),
    in_specs=[
        pl.BlockSpec((MLP_BM, N_CLASSES), lambda i: (i, 0)),
        pl.BlockSpec((MLP_BM, N_CLASSES), lambda i: (i, 0)),
    ],
    out_specs=[pl.BlockSpec((MLP_BM, N_CLASSES), lambda i: (i, 0))] * 5,
    out_shape=[_out16] * 5,
)


def kernel(local_preds, edge_index, W1, W2):
    x = jnp.pad(local_preds, ((0, NP_ - N), (0, 0)))
    pad_idx = jnp.full((NEP - E,), NP_ - 1, jnp.int32)
    row = jnp.concatenate([edge_index[0], pad_idx]).reshape(TILES, CPT, CH)
    col = jnp.concatenate([edge_index[1], pad_idx]).reshape(TILES, CPT, CH)

    local = _local(x, W1, W2)
    deg = _deg_step(row)
    u0, d2, la, d2l, lal = _scales(local, deg)
    preds = _power(u0, col, row, d2, la, d2l, lal)
    return preds[:N]


# speedup vs baseline: 46.9908x; 1.0430x over previous
import functools

import jax
import jax.numpy as jnp
from jax import lax
from jax.experimental import pallas as pl
from jax.experimental.pallas import tpu as pltpu
from jax.experimental.pallas import tpu_sc as plsc

N = 10000
E = 320000
IN_FEATS = 128
N_CLASSES = 16
ALPHA = 0.1
NITER = 10

NP_ = 10240
CH = 128
TILES = 16
CPT = 160
EPT = CPT * CH
NEP = TILES * EPT
RPT = NP_ // 32
RPS = NP_ // TILES
NB = 4
NBATCH = CPT // NB
ZR = 64
MLP_BM = 256
SCALE = 1.0 - ALPHA
SQS = SCALE ** 0.5

_mesh = plsc.VectorSubcoreMesh(core_axis_name="c", subcore_axis_name="s")
_params = pltpu.CompilerParams(use_tc_tiling_on_sc=False)
_out16 = jax.ShapeDtypeStruct((NP_, N_CLASSES), jnp.float32)


def _rsqrt3(x):
    i = lax.bitcast_convert_type(x, jnp.int32)
    i = 0x5F3759DF - lax.shift_right_arithmetic(i, 1)
    y = lax.bitcast_convert_type(i, jnp.float32)
    for _ in range(3):
        y = y * (1.5 - 0.5 * x * y * y)
    return y


@functools.partial(
    pl.kernel,
    out_type=_out16,
    mesh=_mesh,
    scratch_types=[
        pltpu.VMEM_SHARED((NP_, N_CLASSES), jnp.float32),
        pltpu.VMEM_SHARED((NP_, N_CLASSES), jnp.float32),
        pltpu.VMEM((CPT, CH), jnp.int32),
        pltpu.VMEM((CPT, CH), jnp.int32),
        pltpu.VMEM((2 * NB, CH, N_CLASSES), jnp.float32),
        pltpu.VMEM((ZR, N_CLASSES), jnp.float32),
        pltpu.VMEM((RPS, N_CLASSES), jnp.float32),
        pltpu.VMEM((RPS, N_CLASSES), jnp.float32),
        pltpu.VMEM((RPS, N_CLASSES), jnp.float32),
        pltpu.VMEM((RPS, N_CLASSES), jnp.float32),
        pltpu.SemaphoreType.DMA,
        pltpu.SemaphoreType.DMA,
        pltpu.SemaphoreType.DMA,
    ],
    compiler_params=_params,
)
def _power(local_hbm, col_hbm, row_hbm, out_hbm, u_sh, agg_sh, colv, rowv,
           ring, zbuf, agg_v, u_v, d2_v, la_v, semg0, semg1, sems):
    c = lax.axis_index("c")
    s = lax.axis_index("s")
    mine = s * RPS

    pltpu.sync_copy(col_hbm.at[s], colv)
    pltpu.sync_copy(row_hbm.at[s], rowv)
    pltpu.sync_copy(local_hbm.at[pl.ds(mine, RPS)], u_v)

    def zb_fill(i, _):
        zbuf[i] = jnp.zeros((N_CLASSES,), jnp.float32)
        return 0
    lax.fori_loop(0, ZR, zb_fill, 0)

    def one_fill(i, _):
        ring[0, i] = jnp.full((N_CLASSES,), 1.0, jnp.float32)
        return 0
    lax.fori_loop(0, CH, one_fill, 0)

    def _zero_agg():
        def zb(j, _):
            pltpu.sync_copy(zbuf, agg_sh.at[pl.ds(mine + j * ZR, ZR)])
            return 0
        lax.fori_loop(0, RPS // ZR, zb, 0)

    _zero_agg()
    plsc.subcore_barrier()

    def deg_batch(i, _):
        for b in range(NB):
            pltpu.async_copy(ring.at[0], agg_sh.at[rowv.at[i * NB + b]],
                             sems, add=True)

        @pl.when(i >= 1)
        def _():
            for _b in range(NB):
                pltpu.make_async_copy(ring.at[0], agg_sh.at[rowv.at[0]],
                                     sems).wait()
        return 0
    lax.fori_loop(0, NBATCH, deg_batch, 0)
    for _b in range(NB):
        pltpu.make_async_copy(ring.at[0], agg_sh.at[rowv.at[0]], sems).wait()
    plsc.subcore_barrier()

    pltpu.sync_copy(agg_sh.at[pl.ds(mine, RPS)], agg_v)
    _zero_agg()

    def scale_body(r, _):
        deg = agg_v[r] + 1.0
        di = _rsqrt3(deg)
        di = jnp.where(mine + r < N, di, 0.0)
        l = u_v[r]
        u_v[r] = di * l
        la_v[r] = ALPHA * di * l
        d2_v[r] = SCALE * di * di
        return 0
    lax.fori_loop(0, RPS, scale_body, 0)
    pltpu.sync_copy(u_v, u_sh.at[pl.ds(mine, RPS)])
    plsc.subcore_barrier()

    def _fire_gathers(base_k, slot, sem):
        for b in range(NB):
            pltpu.async_copy(u_sh.at[colv.at[base_k + b]],
                             ring.at[slot + b], sem)

    def _drain(n, sem):
        for _b in range(n):
            pltpu.make_async_copy(u_sh.at[colv.at[0]], ring.at[0], sem).wait()

    def one_iter(t, _):
        _fire_gathers(0, 0, semg0)

        def batch(i, _):
            par = i % 2
            h = par * NB
            h2 = NB - h

            @pl.when(i >= 1)
            def _():
                _drain(NB, sems)

            @pl.when((i + 1 < NBATCH) & (par == 0))
            def _():
                _fire_gathers((i + 1) * NB, NB, semg1)

            @pl.when((i + 1 < NBATCH) & (par == 1))
            def _():
                _fire_gathers((i + 1) * NB, 0, semg0)

            @pl.when(par == 0)
            def _():
                _drain(NB, semg0)

            @pl.when(par == 1)
            def _():
                _drain(NB, semg1)

            for b in range(NB):
                pltpu.async_copy(ring.at[h + b],
                                 agg_sh.at[rowv.at[i * NB + b]], sems,
                                 add=True)
            return 0

        lax.fori_loop(0, NBATCH, batch, 0)
        _drain(NB, sems)
        plsc.subcore_barrier()

        pltpu.sync_copy(agg_sh.at[pl.ds(mine, RPS)], agg_v)
        _zero_agg()
        last = t == NITER - 1

        @pl.when(jnp.logical_not(last))
        def _():
            def fin(r, _):
                r4 = r * 4
                for j in range(4):
                    u_v[r4 + j] = (d2_v[r4 + j]
                                   * (agg_v[r4 + j] + u_v[r4 + j])
                                   + la_v[r4 + j])
                return 0
            lax.fori_loop(0, RPS // 4, fin, 0)

        @pl.when(last)
        def _():
            def fin(r, _):
                rs = _rsqrt3(d2_v[r])
                dl = SQS * d2_v[r] * rs
                ll = SQS * la_v[r] * rs
                u_v[r] = dl * (agg_v[r] + u_v[r]) + ll
                return 0
            lax.fori_loop(0, RPS, fin, 0)

        pltpu.sync_copy(u_v, u_sh.at[pl.ds(mine, RPS)])
        plsc.subcore_barrier()
        return 0

    lax.fori_loop(0, NITER, one_iter, 0)

    base = c * (NP_ // 2) + s * RPT
    pltpu.sync_copy(u_sh.at[pl.ds(base, RPT)], agg_v.at[pl.ds(0, RPT)])
    pltpu.sync_copy(agg_v.at[pl.ds(0, RPT)], out_hbm.at[pl.ds(base, RPT)])


def _local_body(x_ref, w1_ref, w2_ref, local_ref):
    h = jnp.tanh(jnp.dot(x_ref[...], w1_ref[...],
                         preferred_element_type=jnp.float32))
    local_ref[...] = jnp.dot(h, w2_ref[...],
                             preferred_element_type=jnp.float32)


_local = pl.pallas_call(
    _local_body,
    grid=(NP_ // MLP_BM,),
    in_specs=[
        pl.BlockSpec((MLP_BM, IN_FEATS), lambda i: (i, 0)),
        pl.BlockSpec((IN_FEATS, 64), lambda i: (0, 0)),
        pl.BlockSpec((64, N_CLASSES), lambda i: (0, 0)),
    ],
    out_specs=pl.BlockSpec((MLP_BM, N_CLASSES), lambda i: (i, 0)),
    out_shape=_out16,
)


def kernel(local_preds, edge_index, W1, W2):
    x = jnp.pad(local_preds, ((0, NP_ - N), (0, 0)))
    pad_idx = jnp.full((NEP - E,), NP_ - 1, jnp.int32)
    row = jnp.concatenate([edge_index[0], pad_idx]).reshape(TILES, CPT, CH)
    col = jnp.concatenate([edge_index[1], pad_idx]).reshape(TILES, CPT, CH)

    local = _local(x, W1, W2)
    preds = _power(local, col, row)
    return preds[:N]
